# Initial kernel scaffold; baseline (speedup 1.0000x reference)
import jax
import jax.numpy as jnp
from jax.experimental import pallas as pl


def kernel(x, Ws, bs, W1, b1, W2, b2, temp):
    raise NotImplementedError("write your pallas kernel here")


# speedup vs baseline: 1.2626x; 1.2626x over previous
import jax
import jax.numpy as jnp
from jax.experimental import pallas as pl
from jax.experimental.pallas import tpu as pltpu

_B = 4
_S = 4096
_DIM = 6144
_NE = 80
_TOPK = 8
_HID = 256
_BALANCE_W = 0.3
_ENTROPY_W = 0.1
_Z_W = 0.0001

_CHUNK = 512
_NCHUNK = _S // _CHUNK


def _router_kernel(x_ref, ws_ref, bs_ref, w1_ref, b1_ref, w2_ref, b2_ref,
                   temp_ref, w_out, i_out, u_out, l_out, acc_ref):
    b = pl.program_id(0)
    i = pl.program_id(1)

    partial = jnp.sum(x_ref[0], axis=0, keepdims=True)

    @pl.when(i == 0)
    def _init():
        acc_ref[pl.ds(b, 1), :] = partial

    @pl.when(i != 0)
    def _acc():
        acc_ref[pl.ds(b, 1), :] += partial

    @pl.when((b == _B - 1) & (i == _NCHUNK - 1))
    def _tail():
        xm = acc_ref[...] * (1.0 / _S)

        h = jax.lax.dot_general(xm, w1_ref[...], (((1,), (1,)), ((), ())),
                                preferred_element_type=jnp.float32)
        h = jnp.maximum(h + b1_ref[...], 0.0)
        pol = jax.lax.dot_general(h, w2_ref[...], (((1,), (1,)), ((), ())),
                                  preferred_element_type=jnp.float32)
        pol = pol + b2_ref[...]
        pol = pol - jnp.max(pol, axis=-1, keepdims=True)
        pol = jnp.exp(pol)
        pol = pol / jnp.sum(pol, axis=-1, keepdims=True)

        t = jnp.maximum(temp_ref[0, 0], 0.1)
        base = jax.lax.dot_general(xm, ws_ref[...], (((1,), (1,)), ((), ())),
                                   preferred_element_type=jnp.float32)
        base = (base + bs_ref[...]) / t

        sc = (base + pol) * 0.5
        sc = sc - jnp.max(sc, axis=-1, keepdims=True)
        sc = jnp.exp(sc)
        scores = sc / jnp.sum(sc, axis=-1, keepdims=True)

        entropy = -jnp.mean(jnp.sum(scores * jnp.log(scores + 1e-6), axis=-1))
        entropy_loss = -_ENTROPY_W * entropy
        bmax = jnp.max(base, axis=-1, keepdims=True)
        lse = jnp.log(jnp.sum(jnp.exp(base - bmax), axis=-1, keepdims=True)) + bmax
        z_loss = _Z_W * jnp.mean(lse * lse)

        lane = jax.lax.broadcasted_iota(jnp.int32, (_B, _NE), 1)
        rem = scores
        usage = jnp.zeros((1, _NE), jnp.float32)
        w_cols = []
        i_cols = []
        for _ in range(_TOPK):
            m = jnp.max(rem, axis=-1, keepdims=True)
            hit = rem == m
            idx = jnp.min(jnp.where(hit, lane, _NE), axis=-1, keepdims=True)
            w = m * t
            w_cols.append(w)
            i_cols.append(idx)
            usage = usage + jnp.sum(jnp.where(lane == idx, w, 0.0), axis=0,
                                    keepdims=True)
            rem = jnp.where(lane == idx, -1.0, rem)

        u_out[...] = usage
        w_out[...] = jnp.concatenate(w_cols, axis=1)
        i_out[...] = jnp.concatenate(i_cols, axis=1)

        frac = usage / (jnp.mean(usage) + 1e-6)
        mu = jnp.mean(frac)
        var = jnp.sum((frac - mu) ** 2) / (_NE - 1)
        loss = _BALANCE_W * var + entropy_loss + z_loss
        l_out[...] = jnp.reshape(loss, (1, 1))


def kernel(x, Ws, bs, W1, b1, W2, b2, temp):
    bs2 = bs.reshape(1, _NE)
    b1_2 = b1.reshape(1, _HID)
    b2_2 = b2.reshape(1, _NE)
    temp2 = jnp.reshape(temp, (1, 1)).astype(jnp.float32)

    fixed = lambda b, i: (0, 0)
    weights, indices, usage, loss = pl.pallas_call(
        _router_kernel,
        grid=(_B, _NCHUNK),
        in_specs=[
            pl.BlockSpec((1, _CHUNK, _DIM), lambda b, i: (b, i, 0)),
            pl.BlockSpec((_NE, _DIM), fixed),
            pl.BlockSpec((1, _NE), fixed),
            pl.BlockSpec((_HID, _DIM), fixed),
            pl.BlockSpec((1, _HID), fixed),
            pl.BlockSpec((_NE, _HID), fixed),
            pl.BlockSpec((1, _NE), fixed),
            pl.BlockSpec((1, 1), fixed),
        ],
        out_specs=[
            pl.BlockSpec((_B, _TOPK), fixed),
            pl.BlockSpec((_B, _TOPK), fixed),
            pl.BlockSpec((1, _NE), fixed),
            pl.BlockSpec((1, 1), fixed),
        ],
        out_shape=[
            jax.ShapeDtypeStruct((_B, _TOPK), jnp.float32),
            jax.ShapeDtypeStruct((_B, _TOPK), jnp.int32),
            jax.ShapeDtypeStruct((1, _NE), jnp.float32),
            jax.ShapeDtypeStruct((1, 1), jnp.float32),
        ],
        scratch_shapes=[pltpu.VMEM((_B, _DIM), jnp.float32)],
    )(x, Ws, bs2, W1, b1_2, W2, b2_2, temp2)

    return (weights, indices, usage.reshape(_NE), loss[0, 0],
            jnp.asarray(0.0, jnp.float32), jnp.asarray(0.0, jnp.float32))
